# Initial kernel scaffold; baseline (speedup 1.0000x reference)
import functools

import jax
import jax.numpy as jnp
from jax import lax
from jax.experimental import pallas as pl
from jax.experimental.pallas import tpu as pltpu
from jax.experimental.pallas import tpu_sc as plsc

_B, _T, _C = 16, 2048, 96
_BLANK = _C - 1
_TBLK = 512
_L = 16
_NC, _NS = 2, 16


def _argmax_body(x_ref, out_ref):
    x = x_ref[0]
    m = jnp.max(x, axis=-1, keepdims=True)
    iot = lax.broadcasted_iota(jnp.int32, x.shape, 1)
    out_ref[0, 0, :] = jnp.min(jnp.where(x == m, iot, _C), axis=-1)


def _tc_argmax(inputs):
    out3 = pl.pallas_call(
        _argmax_body,
        grid=(_B, _T // _TBLK),
        in_specs=[pl.BlockSpec((1, _TBLK, _C), lambda b, t: (b, t, 0))],
        out_specs=pl.BlockSpec((1, 1, _TBLK), lambda b, t: (b, 0, t)),
        out_shape=jax.ShapeDtypeStruct((_B, 1, _T), jnp.int32),
    )(inputs)
    return out3.reshape(_B, _T)


_sc_mesh = plsc.VectorSubcoreMesh(core_axis_name="c", subcore_axis_name="s")


@functools.partial(
    pl.kernel,
    mesh=_sc_mesh,
    out_type=jax.ShapeDtypeStruct((_B, _T), jnp.int32),
    scratch_types=[
        pltpu.VMEM((_T,), jnp.int32),
        pltpu.VMEM((_T,), jnp.int32),
    ],
)
def _sc_decode(preds_hbm, out_hbm, row_v, out_v):
    wid = lax.axis_index("s") * _NC + lax.axis_index("c")

    @pl.when(wid < _B)
    def _():
        pltpu.sync_copy(preds_hbm.at[wid], row_v)
        blank_v = jnp.full((_L,), _BLANK, jnp.int32)

        def fill(i, c):
            out_v[pl.ds(i * _L, _L)] = blank_v
            return c

        lax.fori_loop(0, _T // _L, fill, 0)

        iota = lax.iota(jnp.int32, _L)

        def step(i, carry):
            base = i * _L
            idx = base + iota
            v = row_v[pl.ds(base, _L)]
            p = plsc.load_gather(row_v, [jnp.maximum(idx - 1, 0)])
            keep = ((v != p) | (idx == 0)) & (v != _BLANK)
            ks = plsc.cumsum(keep.astype(jnp.int32))
            pos = jnp.maximum(carry + ks - 1, 0)
            plsc.store_scatter(out_v, [pos], v, mask=keep)
            return carry + plsc.all_reduce_population_count(keep)

        lax.fori_loop(0, _T // _L, step, jnp.zeros((_L,), jnp.int32))
        pltpu.sync_copy(out_v, out_hbm.at[wid])


def kernel(inputs):
    preds = _tc_argmax(inputs)
    out = _sc_decode(preds)
    return out.astype(jnp.int64)


# speedup vs baseline: 1.5716x; 1.5716x over previous
import functools

import jax
import jax.numpy as jnp
from jax import lax
from jax.experimental import pallas as pl
from jax.experimental.pallas import tpu as pltpu
from jax.experimental.pallas import tpu_sc as plsc

_B, _T, _C = 16, 2048, 96
_BLANK = _C - 1
_TBLK = 512
_L = 16
_NC, _NS = 2, 16


def _argmax_body(x_ref, out_ref):
    x = x_ref[0]
    m = jnp.max(x, axis=-1, keepdims=True)
    iot = lax.broadcasted_iota(jnp.int32, x.shape, 1)
    out_ref[0, 0, :] = jnp.min(jnp.where(x == m, iot, _C), axis=-1)


def _tc_argmax(inputs):
    out3 = pl.pallas_call(
        _argmax_body,
        grid=(_B, _T // _TBLK),
        in_specs=[pl.BlockSpec((1, _TBLK, _C), lambda b, t: (b, t, 0))],
        out_specs=pl.BlockSpec((1, 1, _TBLK), lambda b, t: (b, 0, t)),
        out_shape=jax.ShapeDtypeStruct((_B, 1, _T), jnp.int32),
    )(inputs)
    return out3.reshape(_B, _T)


_sc_mesh = plsc.VectorSubcoreMesh(core_axis_name="c", subcore_axis_name="s")


@functools.partial(
    pl.kernel,
    mesh=_sc_mesh,
    out_type=jax.ShapeDtypeStruct((_B, _T), jnp.int32),
    scratch_types=[
        pltpu.VMEM((_T,), jnp.int32),
        pltpu.VMEM((_T,), jnp.int32),
    ],
    compiler_params=pltpu.CompilerParams(needs_layout_passes=False),
)
def _sc_decode(preds_hbm, out_hbm, row_v, out_v):
    wid = lax.axis_index("s") * _NC + lax.axis_index("c")

    @pl.when(wid < _B)
    def _():
        pltpu.sync_copy(preds_hbm.at[wid], row_v)
        blank_v = jnp.full((_L,), _BLANK, jnp.int32)

        def fill(i, c):
            out_v[pl.ds(i * _L, _L)] = blank_v
            return c

        lax.fori_loop(0, _T // _L, fill, 0)

        iota = lax.iota(jnp.int32, _L)

        def step(i, carry):
            base = i * _L
            idx = base + iota
            v = row_v[pl.ds(base, _L)]
            p = plsc.load_gather(row_v, [jnp.maximum(idx - 1, 0)])
            keep = ((v != p) | (idx == 0)) & (v != _BLANK)
            ks = plsc.cumsum(keep.astype(jnp.int32))
            pos = jnp.maximum(carry + ks - 1, 0)
            plsc.store_scatter(out_v, [pos], v, mask=keep)
            return carry + plsc.all_reduce_population_count(keep)

        lax.fori_loop(0, _T // _L, step, jnp.zeros((_L,), jnp.int32))
        pltpu.sync_copy(out_v, out_hbm.at[wid])


def kernel(inputs):
    preds = _tc_argmax(inputs)
    out = _sc_decode(preds)
    return out.astype(jnp.int64)


# speedup vs baseline: 1.6917x; 1.0765x over previous
import functools

import jax
import jax.numpy as jnp
from jax import lax
from jax.experimental import pallas as pl
from jax.experimental.pallas import tpu as pltpu
from jax.experimental.pallas import tpu_sc as plsc

_B, _T, _C = 16, 2048, 96
_BLANK = _C - 1
_TBLK = 512
_L = 16
_NC, _NS = 2, 16


def _argmax_body(x_ref, out_ref):
    x = x_ref[0]
    out_ref[0, 0, :] = jnp.argmax(x, axis=-1).astype(jnp.int32)


def _tc_argmax(inputs):
    out3 = pl.pallas_call(
        _argmax_body,
        grid=(_B, _T // _TBLK),
        in_specs=[pl.BlockSpec((1, _TBLK, _C), lambda b, t: (b, t, 0))],
        out_specs=pl.BlockSpec((1, 1, _TBLK), lambda b, t: (b, 0, t)),
        out_shape=jax.ShapeDtypeStruct((_B, 1, _T), jnp.int32),
    )(inputs)
    return out3.reshape(_B, _T)


_sc_mesh = plsc.VectorSubcoreMesh(core_axis_name="c", subcore_axis_name="s")


@functools.partial(
    pl.kernel,
    mesh=_sc_mesh,
    out_type=jax.ShapeDtypeStruct((_B, _T), jnp.int32),
    scratch_types=[
        pltpu.VMEM((_T,), jnp.int32),
        pltpu.VMEM((_T,), jnp.int32),
    ],
    compiler_params=pltpu.CompilerParams(needs_layout_passes=False),
)
def _sc_decode(preds_hbm, out_hbm, row_v, out_v):
    wid = lax.axis_index("s") * _NC + lax.axis_index("c")

    @pl.when(wid < _B)
    def _():
        pltpu.sync_copy(preds_hbm.at[wid], row_v)
        blank_v = jnp.full((_L,), _BLANK, jnp.int32)

        def fill(i, c):
            out_v[pl.ds(i * _L, _L)] = blank_v
            return c

        lax.fori_loop(0, _T // _L, fill, 0)

        iota = lax.iota(jnp.int32, _L)

        def step(i, carry):
            base = i * _L
            idx = base + iota
            v = row_v[pl.ds(base, _L)]
            p = plsc.load_gather(row_v, [jnp.maximum(idx - 1, 0)])
            keep = ((v != p) | (idx == 0)) & (v != _BLANK)
            ks = plsc.cumsum(keep.astype(jnp.int32))
            pos = jnp.maximum(carry + ks - 1, 0)
            plsc.store_scatter(out_v, [pos], v, mask=keep)
            return carry + plsc.all_reduce_population_count(keep)

        lax.fori_loop(0, _T // _L, step, jnp.zeros((_L,), jnp.int32))
        pltpu.sync_copy(out_v, out_hbm.at[wid])


def kernel(inputs):
    preds = _tc_argmax(inputs)
    out = _sc_decode(preds)
    return out.astype(jnp.int64)


# speedup vs baseline: 5.0009x; 2.9561x over previous
import functools

import jax
import jax.numpy as jnp
from jax import lax
from jax.experimental import pallas as pl
from jax.experimental.pallas import tpu as pltpu
from jax.experimental.pallas import tpu_sc as plsc

_B, _T, _C = 16, 2048, 96
_BLANK = _C - 1
_TBLK = 256
_L = 16
_NC, _NS = 2, 16


def _argmax_body(xt_ref, out_ref):
    for b in range(_B):
        out_ref[b, :] = jnp.argmax(xt_ref[b], axis=0).astype(jnp.int32)


def _tc_argmax(inputs):
    xt = jnp.transpose(inputs, (0, 2, 1))
    return pl.pallas_call(
        _argmax_body,
        grid=(_T // _TBLK,),
        in_specs=[pl.BlockSpec((_B, _C, _TBLK), lambda t: (0, 0, t))],
        out_specs=pl.BlockSpec((_B, _TBLK), lambda t: (0, t)),
        out_shape=jax.ShapeDtypeStruct((_B, _T), jnp.int32),
    )(xt)


_sc_mesh = plsc.VectorSubcoreMesh(core_axis_name="c", subcore_axis_name="s")


@functools.partial(
    pl.kernel,
    mesh=_sc_mesh,
    out_type=jax.ShapeDtypeStruct((_B, _T), jnp.int32),
    scratch_types=[
        pltpu.VMEM((_T,), jnp.int32),
        pltpu.VMEM((_T,), jnp.int32),
    ],
    compiler_params=pltpu.CompilerParams(needs_layout_passes=False),
)
def _sc_decode(preds_hbm, out_hbm, row_v, out_v):
    wid = lax.axis_index("s") * _NC + lax.axis_index("c")

    @pl.when(wid < _B)
    def _():
        pltpu.sync_copy(preds_hbm.at[wid], row_v)
        blank_v = jnp.full((_L,), _BLANK, jnp.int32)

        def fill(i, c):
            out_v[pl.ds(i * _L, _L)] = blank_v
            return c

        lax.fori_loop(0, _T // _L, fill, 0)

        iota = lax.iota(jnp.int32, _L)

        def step(i, carry):
            base = i * _L
            idx = base + iota
            v = row_v[pl.ds(base, _L)]
            p = plsc.load_gather(row_v, [jnp.maximum(idx - 1, 0)])
            keep = ((v != p) | (idx == 0)) & (v != _BLANK)
            ks = plsc.cumsum(keep.astype(jnp.int32))
            pos = jnp.maximum(carry + ks - 1, 0)
            plsc.store_scatter(out_v, [pos], v, mask=keep)
            return carry + plsc.all_reduce_population_count(keep)

        lax.fori_loop(0, _T // _L, step, jnp.zeros((_L,), jnp.int32))
        pltpu.sync_copy(out_v, out_hbm.at[wid])


def kernel(inputs):
    preds = _tc_argmax(inputs)
    out = _sc_decode(preds)
    return out.astype(jnp.int64)
